# baseline (speedup 1.0000x reference)
import jax
import jax.numpy as jnp
from jax import lax
from jax.experimental import pallas as pl
from jax.experimental.pallas import tpu as pltpu
from jax.experimental.pallas import tpu_sc as plsc

_EMBED = 64
_MAXLEN = 200
_BATCH = 4096

_NC = 2
_NS = 16
_NW = _NC * _NS

_TOTAL = _BATCH * _MAXLEN
_PER_W = _TOTAL // _NW
_CHUNK = 128
_NCHUNK = _PER_W // _CHUNK
_LANES = 16


def _emb_body(table_hbm, idx_hbm, pos_hbm, out_hbm, idx_v, rows_v, pos_v, sem):
    wid = lax.axis_index("s") * _NC + lax.axis_index("c")
    base = wid * _PER_W

    pltpu.sync_copy(pos_hbm, pos_v.at[pl.ds(0, _MAXLEN)])
    pltpu.sync_copy(pos_hbm, pos_v.at[pl.ds(_MAXLEN, _MAXLEN)])

    @pl.loop(0, _NCHUNK)
    def _chunk(j):
        off = base + j * _CHUNK
        pltpu.sync_copy(idx_hbm.at[pl.ds(off, _CHUNK)], idx_v)
        pltpu.async_copy(table_hbm.at[idx_v], rows_v, sem).wait()
        poff = lax.rem(j * _CHUNK, _MAXLEN)

        @pl.loop(0, _CHUNK)
        def _row(r):
            pr = poff + r
            for c in range(0, _EMBED, _LANES):
                rows_v[r, pl.ds(c, _LANES)] = (
                    rows_v[r, pl.ds(c, _LANES)] + pos_v[pr, pl.ds(c, _LANES)]
                )

        pltpu.sync_copy(rows_v, out_hbm.at[pl.ds(off, _CHUNK)])


def kernel(inputs, token_table, pos_emb):
    idx = inputs.reshape(_TOTAL).astype(jnp.int32)
    mesh = plsc.VectorSubcoreMesh(core_axis_name="c", subcore_axis_name="s")
    k = pl.kernel(
        _emb_body,
        out_type=jax.ShapeDtypeStruct((_TOTAL, _EMBED), jnp.float32),
        mesh=mesh,
        compiler_params=pltpu.CompilerParams(use_tc_tiling_on_sc=False),
        scratch_types=[
            pltpu.VMEM((_CHUNK,), jnp.int32),
            pltpu.VMEM((_CHUNK, _EMBED), jnp.float32),
            pltpu.VMEM((2 * _MAXLEN, _EMBED), jnp.float32),
            pltpu.SemaphoreType.DMA,
        ],
    )
    out = k(token_table, idx, pos_emb)
    return out.reshape(_BATCH, _MAXLEN, _EMBED)


# speedup vs baseline: 1.2276x; 1.2276x over previous
import jax
import jax.numpy as jnp
from jax import lax
from jax.experimental import pallas as pl
from jax.experimental.pallas import tpu as pltpu
from jax.experimental.pallas import tpu_sc as plsc

_EMBED = 64
_MAXLEN = 200
_BATCH = 4096

_NC = 2
_NS = 16
_NW = _NC * _NS

_TOTAL = _BATCH * _MAXLEN
_PER_W = _TOTAL // _NW
_CHUNK = 128
_NCHUNK = _PER_W // _CHUNK
_LANES = 16
_NBUF = 4
_POS_ROWS = 320


def _emb_body(table_hbm, idx_hbm, pos_hbm, out_hbm, idx_v, pos_v, *rest):
    a_bufs = rest[0:_NBUF]
    b_bufs = rest[_NBUF:2 * _NBUF]
    gsems = rest[2 * _NBUF:3 * _NBUF]
    wsems = rest[3 * _NBUF:4 * _NBUF]

    wid = lax.axis_index("s") * _NC + lax.axis_index("c")
    base = wid * _PER_W

    pltpu.sync_copy(idx_hbm.at[pl.ds(base, _PER_W)], idx_v)
    pltpu.sync_copy(pos_hbm, pos_v.at[pl.ds(0, _MAXLEN)])
    pltpu.sync_copy(
        pos_hbm.at[pl.ds(0, _POS_ROWS - _MAXLEN)],
        pos_v.at[pl.ds(_MAXLEN, _POS_ROWS - _MAXLEN)],
    )

    def _gather(j, b):
        pltpu.async_copy(
            table_hbm.at[idx_v.at[pl.ds(j * _CHUNK, _CHUNK)]], a_bufs[b], gsems[b]
        )

    for b in range(_NBUF):
        _gather(b, b)

    @pl.loop(0, _NCHUNK, step=_NBUF)
    def _step(j0):
        for b in range(_NBUF):
            j = j0 + b
            pltpu.make_async_copy(
                table_hbm.at[idx_v.at[pl.ds(j * _CHUNK, _CHUNK)]],
                a_bufs[b],
                gsems[b],
            ).wait()
            @pl.when(j >= _NBUF)
            def _():
                pltpu.make_async_copy(
                    b_bufs[b],
                    out_hbm.at[pl.ds(base + (j - _NBUF) * _CHUNK, _CHUNK)],
                    wsems[b],
                ).wait()

            poff = lax.rem(j * _CHUNK, _MAXLEN)

            @pl.loop(0, _CHUNK, unroll=4)
            def _row(r):
                pr = poff + r
                for c in range(0, _EMBED, _LANES):
                    b_bufs[b][r, pl.ds(c, _LANES)] = (
                        a_bufs[b][r, pl.ds(c, _LANES)]
                        + pos_v[pr, pl.ds(c, _LANES)]
                    )

            off = base + j * _CHUNK
            pltpu.async_copy(b_bufs[b], out_hbm.at[pl.ds(off, _CHUNK)], wsems[b])

            @pl.when(j < _NCHUNK - _NBUF)
            def _():
                _gather(j + _NBUF, b)

    for b in range(_NBUF):
        j = _NCHUNK - _NBUF + b
        pltpu.make_async_copy(
            b_bufs[b], out_hbm.at[pl.ds(base + j * _CHUNK, _CHUNK)], wsems[b]
        ).wait()


def kernel(inputs, token_table, pos_emb):
    idx = inputs.reshape(_TOTAL).astype(jnp.int32)
    mesh = plsc.VectorSubcoreMesh(core_axis_name="c", subcore_axis_name="s")
    scratch = (
        [pltpu.VMEM((_PER_W,), jnp.int32), pltpu.VMEM((_POS_ROWS, _EMBED), jnp.float32)]
        + [pltpu.VMEM((_CHUNK, _EMBED), jnp.float32) for _ in range(2 * _NBUF)]
        + [pltpu.SemaphoreType.DMA for _ in range(2 * _NBUF)]
    )
    k = pl.kernel(
        _emb_body,
        out_type=jax.ShapeDtypeStruct((_TOTAL, _EMBED), jnp.float32),
        mesh=mesh,
        compiler_params=pltpu.CompilerParams(use_tc_tiling_on_sc=False),
        scratch_types=scratch,
    )
    out = k(token_table, idx, pos_emb)
    return out.reshape(_BATCH, _MAXLEN, _EMBED)


# speedup vs baseline: 1.6086x; 1.3103x over previous
import jax
import jax.numpy as jnp
from jax import lax
from jax.experimental import pallas as pl
from jax.experimental.pallas import tpu as pltpu
from jax.experimental.pallas import tpu_sc as plsc

_EMBED = 64
_MAXLEN = 200
_BATCH = 4096

_NC = 2
_NS = 16
_NW = _NC * _NS

_ROWS_W = _BATCH // _NW
_G0 = 104
_G1 = _MAXLEN - _G0
_LANES = 16
_NBUF = 4


def _emb_body(table_hbm, idx_hbm, pos_hbm, out_hbm, pos_v, *rest):
    ibufs = rest[0:_NBUF]
    abufs = rest[_NBUF:2 * _NBUF]
    isems = rest[2 * _NBUF:3 * _NBUF]
    gsems = rest[3 * _NBUF:4 * _NBUF]
    wsems = rest[4 * _NBUF:5 * _NBUF]

    wid = lax.axis_index("s") * _NC + lax.axis_index("c")
    row0 = wid * _ROWS_W

    pltpu.sync_copy(pos_hbm, pos_v)

    def _idx_dma(j, s):
        pltpu.async_copy(idx_hbm.at[row0 + j], ibufs[s], isems[s])

    def _gathers(j, s):
        pltpu.async_copy(
            table_hbm.at[ibufs[s].at[pl.ds(0, _G0)]],
            abufs[s].at[pl.ds(0, _G0)],
            gsems[s],
        )
        pltpu.async_copy(
            table_hbm.at[ibufs[s].at[pl.ds(_G0, _G1)]],
            abufs[s].at[pl.ds(_G0, _G1)],
            gsems[s],
        )

    def _wait_gathers(s):
        pltpu.make_async_copy(
            table_hbm.at[ibufs[s].at[pl.ds(0, _G0)]],
            abufs[s].at[pl.ds(0, _G0)],
            gsems[s],
        ).wait()
        pltpu.make_async_copy(
            table_hbm.at[ibufs[s].at[pl.ds(_G0, _G1)]],
            abufs[s].at[pl.ds(_G0, _G1)],
            gsems[s],
        ).wait()

    def _wait_idx(s):
        pltpu.make_async_copy(idx_hbm.at[row0], ibufs[s], isems[s]).wait()

    def _wait_wb(s):
        pltpu.make_async_copy(abufs[s], out_hbm.at[row0], wsems[s]).wait()

    for s in range(_NBUF):
        _idx_dma(s, s)
    for jg in range(2):
        _wait_idx(jg)
        _gathers(jg, jg)

    @pl.loop(0, _ROWS_W, step=_NBUF)
    def _steps(j0):
        for b in range(_NBUF):
            j = j0 + b
            sg = (b + 2) % _NBUF

            @pl.when(j < _ROWS_W - 2)
            def _():
                _wait_idx(sg)

                @pl.when(j >= 2)
                def _():
                    _wait_wb(sg)

                _gathers(j + 2, sg)

            _wait_gathers(b)

            @pl.when(j < _ROWS_W - _NBUF)
            def _():
                _idx_dma(j + _NBUF, b)

            @plsc.parallel_loop(0, _MAXLEN, unroll=2)
            def _row(r):
                for c in range(0, _EMBED, _LANES):
                    abufs[b][r, pl.ds(c, _LANES)] = (
                        abufs[b][r, pl.ds(c, _LANES)] + pos_v[r, pl.ds(c, _LANES)]
                    )

            pltpu.async_copy(abufs[b], out_hbm.at[row0 + j], wsems[b])

    for s in range(_NBUF):
        _wait_wb(s)


def kernel(inputs, token_table, pos_emb):
    mesh = plsc.VectorSubcoreMesh(core_axis_name="c", subcore_axis_name="s")
    scratch = (
        [pltpu.VMEM((_MAXLEN, _EMBED), jnp.float32)]
        + [pltpu.VMEM((_MAXLEN,), jnp.int32) for _ in range(_NBUF)]
        + [pltpu.VMEM((_MAXLEN, _EMBED), jnp.float32) for _ in range(_NBUF)]
        + [pltpu.SemaphoreType.DMA for _ in range(3 * _NBUF)]
    )
    k = pl.kernel(
        _emb_body,
        out_type=jax.ShapeDtypeStruct((_BATCH, _MAXLEN, _EMBED), jnp.float32),
        mesh=mesh,
        compiler_params=pltpu.CompilerParams(use_tc_tiling_on_sc=False),
        scratch_types=scratch,
    )
    return k(token_table, inputs.astype(jnp.int32), pos_emb)


# speedup vs baseline: 1.9521x; 1.2135x over previous
import jax
import jax.numpy as jnp
from jax import lax
from jax.experimental import pallas as pl
from jax.experimental.pallas import tpu as pltpu
from jax.experimental.pallas import tpu_sc as plsc

_EMBED = 64
_MAXLEN = 200
_BATCH = 4096
_VOCAB_ROWS = 2000000

_NC = 2
_NS = 16
_NW = _NC * _NS

_ROWS_W = _BATCH // _NW
_G0 = 104
_G1 = _MAXLEN - _G0
_LANES = 16
_NBUF = 4
_TOTAL = _BATCH * _MAXLEN


def _emb_body(table_hbm, idx_hbm, pos_hbm, out_hbm, pos_v, *rest):
    ibufs = rest[0:_NBUF]
    abufs = rest[_NBUF:2 * _NBUF]
    isems = rest[2 * _NBUF:3 * _NBUF]
    gsems = rest[3 * _NBUF:4 * _NBUF]
    wsems = rest[4 * _NBUF:5 * _NBUF]

    wid = lax.axis_index("s") * _NC + lax.axis_index("c")
    row0 = wid * _ROWS_W

    pltpu.sync_copy(pos_hbm, pos_v)

    def _idx_dma(j, s):
        pltpu.async_copy(idx_hbm.at[row0 + j], ibufs[s], isems[s])

    def _gathers(j, s):
        pltpu.async_copy(
            table_hbm.at[ibufs[s].at[pl.ds(0, _G0)]],
            abufs[s].at[pl.ds(0, _G0)],
            gsems[s],
        )
        pltpu.async_copy(
            table_hbm.at[ibufs[s].at[pl.ds(_G0, _G1)]],
            abufs[s].at[pl.ds(_G0, _G1)],
            gsems[s],
        )

    def _wait_gathers(s):
        pltpu.make_async_copy(
            table_hbm.at[ibufs[s].at[pl.ds(0, _G0)]],
            abufs[s].at[pl.ds(0, _G0)],
            gsems[s],
        ).wait()
        pltpu.make_async_copy(
            table_hbm.at[ibufs[s].at[pl.ds(_G0, _G1)]],
            abufs[s].at[pl.ds(_G0, _G1)],
            gsems[s],
        ).wait()

    def _wait_idx(s):
        pltpu.make_async_copy(idx_hbm.at[row0], ibufs[s], isems[s]).wait()

    def _wb(j, s):
        pltpu.async_copy(
            abufs[s],
            out_hbm.at[pl.ds((row0 + j) * _MAXLEN, _MAXLEN), pl.ds(0, _EMBED)],
            wsems[s],
        )

    def _wait_wb(s):
        pltpu.make_async_copy(
            abufs[s],
            out_hbm.at[pl.ds(row0 * _MAXLEN, _MAXLEN), pl.ds(0, _EMBED)],
            wsems[s],
        ).wait()

    for s in range(_NBUF):
        _idx_dma(s, s)
    for jg in range(2):
        _wait_idx(jg)
        _gathers(jg, jg)

    @pl.loop(0, _ROWS_W, step=_NBUF)
    def _steps(j0):
        for b in range(_NBUF):
            j = j0 + b
            sg = (b + 2) % _NBUF

            @pl.when(j < _ROWS_W - 2)
            def _():
                _wait_idx(sg)

                @pl.when(j >= 2)
                def _():
                    _wait_wb(sg)

                _gathers(j + 2, sg)

            _wait_gathers(b)

            @pl.when(j < _ROWS_W - _NBUF)
            def _():
                _idx_dma(j + _NBUF, b)

            @plsc.parallel_loop(0, _MAXLEN, unroll=2)
            def _row(r):
                for c in range(0, _EMBED, _LANES):
                    abufs[b][r, pl.ds(c, _LANES)] = (
                        abufs[b][r, pl.ds(c, _LANES)] + pos_v[r, pl.ds(c, _LANES)]
                    )

            _wb(j, b)

    for s in range(_NBUF):
        _wait_wb(s)


def kernel(inputs, token_table, pos_emb):
    table_pad = jnp.concatenate([token_table, token_table], axis=1)
    table2 = table_pad.reshape(_VOCAB_ROWS, _EMBED)
    idx2 = (inputs * 2).astype(jnp.int32)
    mesh = plsc.VectorSubcoreMesh(core_axis_name="c", subcore_axis_name="s")
    scratch = (
        [pltpu.VMEM((_MAXLEN, _EMBED), jnp.float32)]
        + [pltpu.VMEM((_MAXLEN,), jnp.int32) for _ in range(_NBUF)]
        + [pltpu.VMEM((_MAXLEN, _EMBED), jnp.float32) for _ in range(_NBUF)]
        + [pltpu.SemaphoreType.DMA for _ in range(3 * _NBUF)]
    )
    k = pl.kernel(
        _emb_body,
        out_type=jax.ShapeDtypeStruct((_TOTAL, 128), jnp.float32),
        mesh=mesh,
        compiler_params=pltpu.CompilerParams(use_tc_tiling_on_sc=False),
        scratch_types=scratch,
    )
    out = k(table2, idx2, pos_emb)
    return out[:, :_EMBED].reshape(_BATCH, _MAXLEN, _EMBED)


# speedup vs baseline: 2.1378x; 1.0951x over previous
import jax
import jax.numpy as jnp
from jax import lax
from jax.experimental import pallas as pl
from jax.experimental.pallas import tpu as pltpu
from jax.experimental.pallas import tpu_sc as plsc

_EMBED = 64
_MAXLEN = 200
_BATCH = 4096
_VOCAB = 1000000

_NC = 2
_NS = 16
_NW = _NC * _NS

_ROWS_W = _BATCH // _NW
_G0 = 104
_G1 = _MAXLEN - _G0
_LANES = 16
_NBUF = 4
_TOTAL = _BATCH * _MAXLEN


def _emb_body(table_hbm, idx_hbm, pos_hbm, out_hbm, pos_v, *rest):
    ibufs = rest[0:_NBUF]
    abufs = rest[_NBUF:2 * _NBUF]
    isems = rest[2 * _NBUF:3 * _NBUF]
    gsems = rest[3 * _NBUF:4 * _NBUF]
    wsems = rest[4 * _NBUF:5 * _NBUF]

    wid = lax.axis_index("s") * _NC + lax.axis_index("c")
    row0 = wid * _ROWS_W

    pltpu.sync_copy(pos_hbm, pos_v)

    def _idx_dma(j, s):
        pltpu.async_copy(idx_hbm.at[row0 + j], ibufs[s], isems[s])

    def _gathers(j, s):
        pltpu.async_copy(
            table_hbm.at[ibufs[s].at[pl.ds(0, _G0)]],
            abufs[s].at[pl.ds(0, _G0)],
            gsems[s],
        )
        pltpu.async_copy(
            table_hbm.at[ibufs[s].at[pl.ds(_G0, _G1)]],
            abufs[s].at[pl.ds(_G0, _G1)],
            gsems[s],
        )

    def _wait_gathers(s):
        pltpu.make_async_copy(
            table_hbm.at[ibufs[s].at[pl.ds(0, _G0)]],
            abufs[s].at[pl.ds(0, _G0)],
            gsems[s],
        ).wait()
        pltpu.make_async_copy(
            table_hbm.at[ibufs[s].at[pl.ds(_G0, _G1)]],
            abufs[s].at[pl.ds(_G0, _G1)],
            gsems[s],
        ).wait()

    def _wait_idx(s):
        pltpu.make_async_copy(idx_hbm.at[row0], ibufs[s], isems[s]).wait()

    def _wb(j, s):
        pltpu.async_copy(
            abufs[s],
            out_hbm.at[pl.ds((row0 + j) * _MAXLEN, _MAXLEN), pl.ds(0, _EMBED)],
            wsems[s],
        )

    def _wait_wb(s):
        pltpu.make_async_copy(
            abufs[s],
            out_hbm.at[pl.ds(row0 * _MAXLEN, _MAXLEN), pl.ds(0, _EMBED)],
            wsems[s],
        ).wait()

    for s in range(_NBUF):
        _idx_dma(s, s)
    for jg in range(2):
        _wait_idx(jg)
        _gathers(jg, jg)

    @pl.loop(0, _ROWS_W, step=_NBUF)
    def _steps(j0):
        for b in range(_NBUF):
            j = j0 + b
            sg = (b + 2) % _NBUF

            @pl.when(j < _ROWS_W - 2)
            def _():
                _wait_idx(sg)

                @pl.when(j >= 2)
                def _():
                    _wait_wb(sg)

                _gathers(j + 2, sg)

            _wait_gathers(b)

            @pl.when(j < _ROWS_W - _NBUF)
            def _():
                _idx_dma(j + _NBUF, b)

            @plsc.parallel_loop(0, _MAXLEN, unroll=2)
            def _row(r):
                for c in range(0, _EMBED, _LANES):
                    abufs[b][r, pl.ds(c, _LANES)] = (
                        abufs[b][r, pl.ds(c, _LANES)] + pos_v[r, pl.ds(c, _LANES)]
                    )

            _wb(j, b)

    for s in range(_NBUF):
        _wait_wb(s)


def kernel(inputs, token_table, pos_emb):
    mesh = plsc.VectorSubcoreMesh(core_axis_name="c", subcore_axis_name="s")
    idx = inputs.astype(jnp.int32)
    scratch = (
        [pltpu.VMEM((_MAXLEN, _EMBED), jnp.float32)]
        + [pltpu.VMEM((_MAXLEN,), jnp.int32) for _ in range(_NBUF)]
        + [pltpu.VMEM((_MAXLEN, _EMBED), jnp.float32) for _ in range(_NBUF)]
        + [pltpu.SemaphoreType.DMA for _ in range(3 * _NBUF)]
    )
    k = pl.kernel(
        _emb_body,
        out_type=jax.ShapeDtypeStruct((_TOTAL, 128), jnp.float32),
        mesh=mesh,
        compiler_params=pltpu.CompilerParams(use_tc_tiling_on_sc=False),
        scratch_types=scratch,
    )
    out = k(token_table, idx, pos_emb)
    return out[:, :_EMBED].reshape(_BATCH, _MAXLEN, _EMBED)
